# initial kernel scaffold (unmeasured)
import jax
import jax.numpy as jnp
from jax import lax
from jax.experimental import pallas as pl
from jax.experimental.pallas import tpu as pltpu

N_DEV = 16


def kernel(x, w_mat):
    m_total, k_my = x.shape
    k_total, n = w_mat.shape
    m_blk = m_total // N_DEV
    assert k_total == N_DEV * k_my

    def body(x_ref, w_ref, out_ref, buf_ref, send_sems, recv_sems):
        me = lax.axis_index("i")

        barrier_sem = pltpu.get_barrier_semaphore()
        for d in range(1, N_DEV):
            peer = lax.rem(me + d, N_DEV)
            pl.semaphore_signal(
                barrier_sem, inc=1,
                device_id=(peer,), device_id_type=pl.DeviceIdType.MESH,
            )
        pl.semaphore_wait(barrier_sem, N_DEV - 1)

        buf_ref[me] = x_ref[pl.ds(me * m_blk, m_blk), :]

        sends = []
        for d in range(1, N_DEV):
            dst = lax.rem(me + d, N_DEV)
            rdma = pltpu.make_async_remote_copy(
                src_ref=x_ref.at[pl.ds(dst * m_blk, m_blk), :],
                dst_ref=buf_ref.at[me],
                send_sem=send_sems.at[d - 1],
                recv_sem=recv_sems.at[me],
                device_id=(dst,),
                device_id_type=pl.DeviceIdType.MESH,
            )
            rdma.start()
            sends.append(rdma)

        for d in range(1, N_DEV):
            src = lax.rem(me + N_DEV - d, N_DEV)
            recv = pltpu.make_async_remote_copy(
                src_ref=x_ref.at[pl.ds(0, m_blk), :],
                dst_ref=buf_ref.at[src],
                send_sem=send_sems.at[d - 1],
                recv_sem=recv_sems.at[src],
                device_id=(me,),
                device_id_type=pl.DeviceIdType.MESH,
            )
            recv.wait_recv()

        acc = jnp.zeros((m_blk, n), jnp.float32)
        for j in range(N_DEV):
            acc += jnp.dot(
                buf_ref[j],
                w_ref[j * k_my:(j + 1) * k_my, :],
                preferred_element_type=jnp.float32,
            )

        c = 0.7978845608028654
        out_ref[:, :] = 0.5 * acc * (1.0 + jnp.tanh(c * (acc + 0.044715 * acc * acc * acc)))

        for rdma in sends:
            rdma.wait_send()

    return pl.pallas_call(
        body,
        out_shape=jax.ShapeDtypeStruct((m_blk, n), jnp.float32),
        in_specs=[
            pl.BlockSpec(memory_space=pltpu.VMEM),
            pl.BlockSpec(memory_space=pltpu.VMEM),
        ],
        out_specs=pl.BlockSpec(memory_space=pltpu.VMEM),
        scratch_shapes=[
            pltpu.VMEM((N_DEV, m_blk, k_my), x.dtype),
            pltpu.SemaphoreType.DMA((N_DEV - 1,)),
            pltpu.SemaphoreType.DMA((N_DEV,)),
        ],
        compiler_params=pltpu.CompilerParams(collective_id=0),
    )(x, w_mat)


# baseline (device time: 56066 ns/iter reference)
import jax
import jax.numpy as jnp
from jax import lax
from jax.experimental import pallas as pl
from jax.experimental.pallas import tpu as pltpu

N_DEV = 16


def kernel(x, w_mat):
    m_total, k_my = x.shape
    k_total, n = w_mat.shape
    m_blk = m_total // N_DEV
    assert k_total == N_DEV * k_my

    def body(x_ref, w_hbm, out_ref, buf_ref, send_ref, w_slots,
             send_sems, recv_sems, wdma_sems):
        me = lax.axis_index("i")

        barrier_sem = pltpu.get_barrier_semaphore()
        for d in range(1, N_DEV):
            peer = lax.rem(me + d, N_DEV)
            pl.semaphore_signal(
                barrier_sem, inc=1,
                device_id=(peer,), device_id_type=pl.DeviceIdType.MESH,
            )
        pl.semaphore_wait(barrier_sem, N_DEV - 1)

        send_ref[:, :] = x_ref[:, :].astype(jnp.bfloat16)
        buf_ref[me] = send_ref[pl.ds(me * m_blk, m_blk), :]

        sends = []
        for d in range(1, N_DEV):
            dst = lax.rem(me + d, N_DEV)
            rdma = pltpu.make_async_remote_copy(
                src_ref=send_ref.at[pl.ds(dst * m_blk, m_blk), :],
                dst_ref=buf_ref.at[me],
                send_sem=send_sems.at[d - 1],
                recv_sem=recv_sems.at[me],
                device_id=(dst,),
                device_id_type=pl.DeviceIdType.MESH,
            )
            rdma.start()
            sends.append(rdma)

        def w_dma(u):
            src = lax.rem(me + N_DEV - (u % N_DEV), N_DEV)
            return pltpu.make_async_copy(
                w_hbm.at[pl.ds(src * k_my, k_my), :],
                w_slots.at[u % 2],
                wdma_sems.at[u % 2],
            )

        w_dma(0).start()
        for u in range(N_DEV):
            if u + 1 < N_DEV:
                w_dma(u + 1).start()
            w_dma(u).wait()

            src = lax.rem(me + N_DEV - u, N_DEV)
            if u > 0:
                recv = pltpu.make_async_remote_copy(
                    src_ref=send_ref.at[pl.ds(0, m_blk), :],
                    dst_ref=buf_ref.at[src],
                    send_sem=send_sems.at[0],
                    recv_sem=recv_sems.at[src],
                    device_id=(me,),
                    device_id_type=pl.DeviceIdType.MESH,
                )
                recv.wait_recv()

            contrib = jnp.dot(
                buf_ref[src],
                w_slots[u % 2].astype(jnp.bfloat16),
                preferred_element_type=jnp.float32,
            )
            if u == 0:
                out_ref[:, :] = contrib
            elif u < N_DEV - 1:
                out_ref[:, :] = out_ref[:, :] + contrib
            else:
                y = out_ref[:, :] + contrib
                c = 0.7978845608028654
                out_ref[:, :] = 0.5 * y * (
                    1.0 + jnp.tanh(c * (y + 0.044715 * y * y * y))
                )

        for rdma in sends:
            rdma.wait_send()

    return pl.pallas_call(
        body,
        out_shape=jax.ShapeDtypeStruct((m_blk, n), jnp.float32),
        in_specs=[
            pl.BlockSpec(memory_space=pltpu.VMEM),
            pl.BlockSpec(memory_space=pltpu.MemorySpace.HBM),
        ],
        out_specs=pl.BlockSpec(memory_space=pltpu.VMEM),
        scratch_shapes=[
            pltpu.VMEM((N_DEV, m_blk, k_my), jnp.bfloat16),
            pltpu.VMEM((m_total, k_my), jnp.bfloat16),
            pltpu.VMEM((2, k_my, n), jnp.float32),
            pltpu.SemaphoreType.DMA((N_DEV - 1,)),
            pltpu.SemaphoreType.DMA((N_DEV,)),
            pltpu.SemaphoreType.DMA((2,)),
        ],
        compiler_params=pltpu.CompilerParams(collective_id=0),
    )(x, w_mat)
